# baseline (speedup 1.0000x reference)
import functools

import jax
import jax.numpy as jnp
from jax import lax
from jax.experimental import pallas as pl
from jax.experimental.pallas import tpu as pltpu
from jax.experimental.pallas import tpu_sc as plsc

RADIUS2 = 25.0
BIG = 1e30
L = 16


def _sc_body(pred_hbm, gt_hbm, out_hbm, pbuf, tbuf, table, lrow, mb, orow,
             sh, *, n_p, n_groups):
    c = lax.axis_index("c")
    s = lax.axis_index("s")
    b = 2 * c + s // 8
    sg = s % 8

    pltpu.sync_copy(pred_hbm.at[b], pbuf)
    pltpu.sync_copy(gt_hbm.at[b, sg], tbuf)

    tcv = tbuf[0, :]
    txv = tbuf[1, :]
    tyv = tbuf[2, :]

    lane = lax.broadcasted_iota(jnp.int32, (L,), 0)

    TC_CHUNK = 8
    UNROLL = 5

    def scan_targets(tslice):
        tb = [jnp.full((L,), t, jnp.int32) for t in tslice]
        txb = [txv.at[i].get(mode="promise_in_bounds") for i in tb]
        tyb = [tyv.at[i].get(mode="promise_in_bounds") for i in tb]
        tcb = [tcv.at[i].get(mode="promise_in_bounds") for i in tb]

        def tile_step(i, carry):
            accs = list(carry)
            for u in range(UNROLL):
                base = (UNROLL * i + u) * L
                pxv = pbuf[1, pl.ds(base, L)]
                pyv = pbuf[2, pl.ds(base, L)]
                pcv = pbuf[0, pl.ds(base, L)]
                idxv = jnp.full((L,), base, jnp.int32) + lane
                for k in range(len(tslice)):
                    bst, bix = accs[2 * k], accs[2 * k + 1]
                    dx = pxv - txb[k]
                    dy = pyv - tyb[k]
                    d2 = dx * dx + dy * dy
                    key = jnp.where(pcv == tcb[k], d2, BIG)
                    lt = key < bst
                    accs[2 * k] = jnp.where(lt, key, bst)
                    accs[2 * k + 1] = jnp.where(lt, idxv, bix)
            return tuple(accs)

        init = []
        for _ in tslice:
            init += [jnp.full((L,), BIG, jnp.float32), jnp.zeros((L,), jnp.int32)]
        accs = lax.fori_loop(0, n_p // (L * UNROLL), tile_step, tuple(init))

        outs = []
        for k in range(len(tslice)):
            bst, bix = accs[2 * k], accs[2 * k + 1]
            for step in (8, 4, 2, 1):
                perm = jnp.bitwise_and(lane + step, L - 1)
                pd2 = bst.at[perm].get(mode="promise_in_bounds")
                pix = bix.at[perm].get(mode="promise_in_bounds")
                better = jnp.logical_or(
                    pd2 < bst, jnp.logical_and(pd2 == bst, pix < bix))
                bst = jnp.where(better, pd2, bst)
                bix = jnp.where(better, pix, bix)
            outs.append((bst, bix))
        return outs

    best = jnp.full((L,), BIG, jnp.float32)
    bidx = jnp.zeros((L,), jnp.int32)
    for h in range(0, L, TC_CHUNK):
        for k, (bst, bix) in enumerate(scan_targets(range(h, h + TC_CHUNK))):
            sel = lane == (h + k)
            best = jnp.where(sel, bst, best)
            bidx = jnp.where(sel, bix, bidx)

    matched = best <= RADIUS2
    lrow[0, :] = jnp.where(matched, 1.0, 0.0)
    lrow[1, :] = bidx.astype(jnp.float32)
    pltpu.sync_copy(lrow, sh.at[2 * L + s])
    plsc.subcore_barrier()

    @pl.when(sg == 0)
    def _dedup():
        pltpu.sync_copy(sh.at[pl.ds(2 * L, L)], mb)
        srow = (s // 8) * 8
        tp = jnp.zeros((L,), jnp.int32)
        lane = lax.broadcasted_iota(jnp.int32, (L,), 0)
        masks = [mb[srow + r, 0, :] > 0.5 for r in range(n_groups)]
        idxs = [mb[srow + r, 1, :].astype(jnp.int32) for r in range(n_groups)]
        tags = [jnp.full((L,), r * L, jnp.int32) + lane for r in range(n_groups)]
        for r in range(n_groups):
            plsc.store_scatter(table, [idxs[r]], tags[r], mask=masks[r])
        for r in range(n_groups):
            g = plsc.load_gather(table, [idxs[r]], mask=masks[r])
            won = jnp.logical_and(masks[r], g == tags[r])
            tp = tp + plsc.all_reduce_population_count(won)
        orow[...] = tp.astype(jnp.float32)
        pltpu.sync_copy(orow, out_hbm.at[b])


def kernel(pred, gt):
    B, P, _ = pred.shape
    T = gt.shape[1]
    TPAD = 128
    n_groups = TPAD // L

    pred_t = jnp.transpose(pred, (0, 2, 1))
    gt_pad = jnp.pad(gt, ((0, 0), (0, TPAD - T), (0, 0)), constant_values=-1.0)
    gt_t = jnp.transpose(gt_pad, (0, 2, 1)).reshape(B, 3, n_groups, L)
    gt_t = jnp.transpose(gt_t, (0, 2, 1, 3))

    mesh = plsc.VectorSubcoreMesh(core_axis_name="c", subcore_axis_name="s")
    body = functools.partial(_sc_body, n_p=P, n_groups=n_groups)
    tp_rows = pl.kernel(
        body,
        out_type=jax.ShapeDtypeStruct((B, L), jnp.float32),
        mesh=mesh,
        compiler_params=pltpu.CompilerParams(needs_layout_passes=False),
        scratch_types=[
            pltpu.VMEM((3, P), jnp.float32),
            pltpu.VMEM((3, L), jnp.float32),
            pltpu.VMEM((P,), jnp.int32),
            pltpu.VMEM((2, L), jnp.float32),
            pltpu.VMEM((L, 2, L), jnp.float32),
            pltpu.VMEM((L,), jnp.float32),
            pltpu.VMEM_SHARED((3 * L, 2, L), jnp.float32),
        ],
    )(pred_t, gt_t)

    tp = jnp.sum(tp_rows[:, 0])
    fp = jnp.float32(B * P) - tp
    fn = jnp.float32(B * T) - tp
    prec = (tp + 1e-06) / (tp + 1e-06 + fp + 1e-06)
    rec = (tp + 1e-06) / (tp + fn + 1e-06)
    f1 = 2.0 * prec * rec / (prec + rec)
    return 1.0 - f1


# speedup vs baseline: 2.3781x; 2.3781x over previous
import functools

import jax
import jax.numpy as jnp
from jax import lax
from jax.experimental import pallas as pl
from jax.experimental.pallas import tpu as pltpu
from jax.experimental.pallas import tpu_sc as plsc

RADIUS2 = 25.0
BIG = 1e30
L = 16


def _sc_body(pred_hbm, gt_hbm, out_hbm, pbuf, tbuf, table, lrow, mb, orow,
             sh, *, n_p, n_groups):
    c = lax.axis_index("c")
    s = lax.axis_index("s")
    b = 2 * c + s // 8
    sg = s % 8

    pltpu.sync_copy(pred_hbm.at[b], pbuf)
    pltpu.sync_copy(gt_hbm.at[b, sg], tbuf)

    tcv = tbuf[0, :]
    txv = tbuf[1, :]
    tyv = tbuf[2, :]

    lane = lax.broadcasted_iota(jnp.int32, (L,), 0)

    TC_CHUNK = 8
    UNROLL = 2

    def scan_targets(tslice):
        tb = [jnp.full((L,), t, jnp.int32) for t in tslice]
        txb = [txv.at[i].get(mode="promise_in_bounds") for i in tb]
        tyb = [tyv.at[i].get(mode="promise_in_bounds") for i in tb]
        tcb = [tcv.at[i].get(mode="promise_in_bounds") for i in tb]

        def tile_step(i, carry):
            accs = list(carry)
            for u in range(UNROLL):
                base = (UNROLL * i + u) * L
                pxv = pbuf[1, pl.ds(base, L)]
                pyv = pbuf[2, pl.ds(base, L)]
                pcv = pbuf[0, pl.ds(base, L)]
                idxv = jnp.full((L,), base, jnp.int32) + lane
                for k in range(len(tslice)):
                    bst, bix = accs[2 * k], accs[2 * k + 1]
                    dx = pxv - txb[k]
                    dy = pyv - tyb[k]
                    d2 = dx * dx + dy * dy
                    key = jnp.where(pcv == tcb[k], d2, BIG)
                    lt = key < bst
                    accs[2 * k] = jnp.where(lt, key, bst)
                    accs[2 * k + 1] = jnp.where(lt, idxv, bix)
            return tuple(accs)

        init = []
        for _ in tslice:
            init += [jnp.full((L,), BIG, jnp.float32), jnp.zeros((L,), jnp.int32)]
        accs = lax.fori_loop(0, n_p // (L * UNROLL), tile_step, tuple(init))

        outs = []
        for k in range(len(tslice)):
            bst, bix = accs[2 * k], accs[2 * k + 1]
            for step in (8, 4, 2, 1):
                perm = jnp.bitwise_and(lane + step, L - 1)
                pd2 = bst.at[perm].get(mode="promise_in_bounds")
                pix = bix.at[perm].get(mode="promise_in_bounds")
                better = jnp.logical_or(
                    pd2 < bst, jnp.logical_and(pd2 == bst, pix < bix))
                bst = jnp.where(better, pd2, bst)
                bix = jnp.where(better, pix, bix)
            outs.append((bst, bix))
        return outs

    best = jnp.full((L,), BIG, jnp.float32)
    bidx = jnp.zeros((L,), jnp.int32)
    for h in range(0, L, TC_CHUNK):
        for k, (bst, bix) in enumerate(scan_targets(range(h, h + TC_CHUNK))):
            sel = lane == (h + k)
            best = jnp.where(sel, bst, best)
            bidx = jnp.where(sel, bix, bidx)

    matched = best <= RADIUS2
    lrow[0, :] = jnp.where(matched, 1.0, 0.0)
    lrow[1, :] = bidx.astype(jnp.float32)
    pltpu.sync_copy(lrow, sh.at[2 * L + s])
    plsc.subcore_barrier()

    @pl.when(sg == 0)
    def _dedup():
        pltpu.sync_copy(sh.at[pl.ds(2 * L, L)], mb)
        srow = (s // 8) * 8
        tp = jnp.zeros((L,), jnp.int32)
        lane = lax.broadcasted_iota(jnp.int32, (L,), 0)
        masks = [mb[srow + r, 0, :] > 0.5 for r in range(n_groups)]
        idxs = [mb[srow + r, 1, :].astype(jnp.int32) for r in range(n_groups)]
        tags = [jnp.full((L,), r * L, jnp.int32) + lane for r in range(n_groups)]
        for r in range(n_groups):
            plsc.store_scatter(table, [idxs[r]], tags[r], mask=masks[r])
        for r in range(n_groups):
            g = plsc.load_gather(table, [idxs[r]], mask=masks[r])
            won = jnp.logical_and(masks[r], g == tags[r])
            tp = tp + plsc.all_reduce_population_count(won)
        orow[...] = tp.astype(jnp.float32)
        pltpu.sync_copy(orow, out_hbm.at[b])


def kernel(pred, gt):
    B, P, _ = pred.shape
    T = gt.shape[1]
    TPAD = 128
    n_groups = TPAD // L

    pred_t = jnp.transpose(pred, (0, 2, 1))
    gt_pad = jnp.pad(gt, ((0, 0), (0, TPAD - T), (0, 0)), constant_values=-1.0)
    gt_t = jnp.transpose(gt_pad, (0, 2, 1)).reshape(B, 3, n_groups, L)
    gt_t = jnp.transpose(gt_t, (0, 2, 1, 3))

    mesh = plsc.VectorSubcoreMesh(core_axis_name="c", subcore_axis_name="s")
    body = functools.partial(_sc_body, n_p=P, n_groups=n_groups)
    tp_rows = pl.kernel(
        body,
        out_type=jax.ShapeDtypeStruct((B, L), jnp.float32),
        mesh=mesh,
        compiler_params=pltpu.CompilerParams(needs_layout_passes=False),
        scratch_types=[
            pltpu.VMEM((3, P), jnp.float32),
            pltpu.VMEM((3, L), jnp.float32),
            pltpu.VMEM((P,), jnp.int32),
            pltpu.VMEM((2, L), jnp.float32),
            pltpu.VMEM((L, 2, L), jnp.float32),
            pltpu.VMEM((L,), jnp.float32),
            pltpu.VMEM_SHARED((3 * L, 2, L), jnp.float32),
        ],
    )(pred_t, gt_t)

    tp = jnp.sum(tp_rows[:, 0])
    fp = jnp.float32(B * P) - tp
    fn = jnp.float32(B * T) - tp
    prec = (tp + 1e-06) / (tp + 1e-06 + fp + 1e-06)
    rec = (tp + 1e-06) / (tp + fn + 1e-06)
    f1 = 2.0 * prec * rec / (prec + rec)
    return 1.0 - f1
